# baseline (speedup 1.0000x reference)
import jax
import jax.numpy as jnp
from jax import lax
from jax.experimental import pallas as pl
from jax.experimental.pallas import tpu as pltpu
from jax.experimental.pallas import tpu_sc as plsc

NC = 2
NS = 16
NW = NC * NS

D = 128
NB = 6


def _build(S, N, V):
    npw = N // NW

    mesh = plsc.VectorSubcoreMesh(core_axis_name="c", subcore_axis_name="s")

    def body(xt_hbm, table_hbm, out_hbm, idx_v, buf, table_sh, gsems, osems):
        wid = lax.axis_index("s") * NC + lax.axis_index("c")
        n0 = wid * npw
        @pl.when(lax.axis_index("s") == 0)
        def _():
            pltpu.sync_copy(table_hbm, table_sh)

        pltpu.sync_copy(xt_hbm.at[:, pl.ds(n0, npw)], idx_v)
        plsc.subcore_barrier()
        for p in range(NB - 1):
            pltpu.async_copy(table_sh.at[idx_v.at[p]], buf.at[p], gsems.at[p])

        def step(j, carry):
            b = lax.rem(j, NB)

            pltpu.make_async_copy(
                table_sh.at[idx_v.at[j]], buf.at[b], gsems.at[b]
            ).wait()
            pltpu.async_copy(
                buf.at[b], out_hbm.at[j, pl.ds(n0, npw)], osems.at[b]
            )

            nb = lax.rem(b + NB - 1, NB)

            @pl.when(j + NB - 1 < S)
            def _():
                @pl.when(j >= 1)
                def _():
                    pltpu.make_async_copy(
                        buf.at[nb], out_hbm.at[0, pl.ds(0, npw)], osems.at[nb]
                    ).wait()

                pltpu.async_copy(
                    table_sh.at[idx_v.at[j + NB - 1]], buf.at[nb], gsems.at[nb]
                )

            return carry

        lax.fori_loop(0, S, step, 0)
        for b in range(NB):
            pltpu.make_async_copy(
                buf.at[b], out_hbm.at[0, pl.ds(0, npw)], osems.at[b]
            ).wait()

    grid_kernel = pl.kernel(
        body,
        out_type=jax.ShapeDtypeStruct((S, N, D), jnp.float32),
        mesh=mesh,
        scratch_types=[
            pltpu.VMEM((S, npw), jnp.int32),
            pltpu.VMEM((NB, npw, D), jnp.float32),
            pltpu.VMEM_SHARED((V, D), jnp.float32),
            pltpu.SemaphoreType.DMA((NB,)),
            pltpu.SemaphoreType.DMA((NB,)),
        ],
    )
    return grid_kernel


def kernel(x, embedding):
    N, S = x.shape
    out_phys = _build(S, N, embedding.shape[0])(x.T, embedding)
    return out_phys.transpose(1, 0, 2)


# speedup vs baseline: 1.0019x; 1.0019x over previous
import jax
import jax.numpy as jnp
from jax import lax
from jax.experimental import pallas as pl
from jax.experimental.pallas import tpu as pltpu
from jax.experimental.pallas import tpu_sc as plsc

NC = 2
NS = 16
NW = NC * NS

D = 128
NB = 4


def _build(S, N, V):
    npw = N // NW

    mesh = plsc.VectorSubcoreMesh(core_axis_name="c", subcore_axis_name="s")

    def body(xt_hbm, table_hbm, out_hbm, idx_v, buf, table_sh, gsems, osems):
        wid = lax.axis_index("s") * NC + lax.axis_index("c")
        n0 = wid * npw
        @pl.when(lax.axis_index("s") == 0)
        def _():
            pltpu.sync_copy(table_hbm, table_sh)

        pltpu.sync_copy(xt_hbm.at[:, pl.ds(n0, npw)], idx_v)
        plsc.subcore_barrier()
        for p in range(NB - 1):
            pltpu.async_copy(table_sh.at[idx_v.at[p]], buf.at[p], gsems.at[p])

        def step(j, carry):
            b = lax.rem(j, NB)

            pltpu.make_async_copy(
                table_sh.at[idx_v.at[j]], buf.at[b], gsems.at[b]
            ).wait()
            pltpu.async_copy(
                buf.at[b], out_hbm.at[j, pl.ds(n0, npw)], osems.at[b]
            )

            nb = lax.rem(b + NB - 1, NB)

            @pl.when(j + NB - 1 < S)
            def _():
                @pl.when(j >= 1)
                def _():
                    pltpu.make_async_copy(
                        buf.at[nb], out_hbm.at[0, pl.ds(0, npw)], osems.at[nb]
                    ).wait()

                pltpu.async_copy(
                    table_sh.at[idx_v.at[j + NB - 1]], buf.at[nb], gsems.at[nb]
                )

            return carry

        lax.fori_loop(0, S, step, 0)
        for b in range(NB):
            pltpu.make_async_copy(
                buf.at[b], out_hbm.at[0, pl.ds(0, npw)], osems.at[b]
            ).wait()

    grid_kernel = pl.kernel(
        body,
        out_type=jax.ShapeDtypeStruct((S, N, D), jnp.float32),
        mesh=mesh,
        scratch_types=[
            pltpu.VMEM((S, npw), jnp.int32),
            pltpu.VMEM((NB, npw, D), jnp.float32),
            pltpu.VMEM_SHARED((V, D), jnp.float32),
            pltpu.SemaphoreType.DMA((NB,)),
            pltpu.SemaphoreType.DMA((NB,)),
        ],
    )
    return grid_kernel


def kernel(x, embedding):
    N, S = x.shape
    out_phys = _build(S, N, embedding.shape[0])(x.T, embedding)
    return out_phys.transpose(1, 0, 2)
